# baseline (speedup 1.0000x reference)
import functools

import jax
import jax.numpy as jnp
from jax import lax
from jax.experimental import pallas as pl
from jax.experimental.pallas import tpu as pltpu
from jax.experimental.pallas import tpu_sc as plsc

N = 10000
K = 32
H = 128
L = 6
NGRAPH = 16

BR = 64
NP = 10240
NT = NP // 128


def _knn_body(cs_ref, ct_ref, rows_ref, rowsb_ref, cols_ref, idx_ref, kscr):
    i = pl.program_id(0)
    c_start = cs_ref[i]
    n_t = ct_ref[i]

    xr = rows_ref[:, 0:1]
    yr = rows_ref[:, 1:2]
    zr = rows_ref[:, 2:3]
    sqr = rows_ref[:, 3:4]
    rbf = rowsb_ref[...].astype(jnp.float32)
    rif = (i * BR + jax.lax.broadcasted_iota(jnp.int32, (BR, 1), 0)
           ).astype(jnp.float32)

    inf = jnp.float32(jnp.inf)

    def dist_tile(t, _):
        tt = c_start + t
        c = cols_ref[pl.ds(tt, 1)][0]
        xc, yc, zc, sqc, bc = c[0:1], c[1:2], c[2:3], c[3:4], c[4:5]
        dot = xr * xc + yr * yc + zr * zc
        d = sqr + sqc - 2.0 * dot
        colf = (tt * 128
                + jax.lax.broadcasted_iota(jnp.int32, (1, 128), 1)
                ).astype(jnp.float32)
        msk = (bc != rbf) | (colf == rif)
        d = jnp.where(msk, inf, d)
        b = d.view(jnp.int32)
        key = b ^ ((b >> 31) & jnp.int32(0x7FFFFFFF))
        kscr[pl.ds(t, 1)] = key[None]
        return 0

    jax.lax.fori_loop(0, n_t, dist_tile, 0)

    ones_col = jnp.ones((128, 1), jnp.float32)

    def bis(it, carry):
        lo, hi = carry
        mid = (lo >> 1) + (hi >> 1) + (lo & hi & 1)

        def cnt_tile(t, acc):
            k = kscr[pl.ds(t, 1)][0]
            return acc + jnp.where(k <= mid, 1.0, 0.0)

        accl = jax.lax.fori_loop(0, n_t, cnt_tile,
                                 jnp.zeros((BR, 128), jnp.float32))
        cnt = jnp.dot(accl, ones_col, preferred_element_type=jnp.float32)
        pick = cnt >= jnp.float32(K)
        return (jnp.where(pick, lo, mid + 1), jnp.where(pick, mid, hi))

    lo0 = jnp.full((BR, 1), jnp.int32(-2**31))
    hi0 = jnp.full((BR, 1), jnp.int32(2**31 - 1))
    lo, hi = jax.lax.fori_loop(0, 32, bis, (lo0, hi0))
    tstar = hi

    def cntlt_tile(t, acc):
        k = kscr[pl.ds(t, 1)][0]
        return acc + jnp.where(k < tstar, 1.0, 0.0)

    clt = jax.lax.fori_loop(0, n_t, cntlt_tile,
                            jnp.zeros((BR, 128), jnp.float32))
    quota = jnp.float32(K) - jnp.dot(clt, ones_col,
                                     preferred_element_type=jnp.float32)

    tri = (jax.lax.broadcasted_iota(jnp.int32, (128, 128), 0)
           <= jax.lax.broadcasted_iota(jnp.int32, (128, 128), 1)
           ).astype(jnp.float32)
    slot_iota = jax.lax.broadcasted_iota(jnp.int32, (1, K), 1)

    def ext_tile(t, carry):
        acc, ce, cc = carry
        tt = c_start + t
        k = kscr[pl.ds(t, 1)][0]
        m_eq_f = jnp.where(k == tstar, 1.0, 0.0)
        peq = jnp.dot(m_eq_f, tri, preferred_element_type=jnp.float32)
        chosen_f = jnp.where(k < tstar, 1.0,
                             m_eq_f * jnp.where(peq + ce <= quota, 1.0, 0.0))
        rank = jnp.dot(chosen_f, tri,
                       preferred_element_type=jnp.float32) + cc
        lanef = jax.lax.broadcasted_iota(jnp.int32, (BR, 128), 1
                                         ).astype(jnp.float32)
        rank_sel = rank * chosen_f
        base_f = jnp.float32(1.0) * (tt * 128)
        for s in range(K):
            m_s = jnp.where(rank_sel == jnp.float32(s + 1), 1.0, 0.0)
            cnt_s = jnp.dot(m_s, ones_col,
                            preferred_element_type=jnp.float32)
            lsum = jnp.dot(m_s * lanef, ones_col,
                           preferred_element_type=jnp.float32)
            contrib = base_f * cnt_s + lsum
            acc = acc + contrib * (slot_iota == s).astype(jnp.float32)
        ce = ce + jnp.dot(m_eq_f, ones_col,
                          preferred_element_type=jnp.float32)
        cc = cc + jnp.dot(chosen_f, ones_col,
                          preferred_element_type=jnp.float32)
        return acc, ce, cc

    acc0 = jnp.zeros((BR, K), jnp.float32)
    z = jnp.zeros((BR, 1), jnp.float32)
    acc, _, _ = jax.lax.fori_loop(0, n_t, ext_tile, (acc0, z, z))
    idx_ref[...] = acc.astype(jnp.int32)


def _knn_call(cs, ct, rows, rowsb, cols):
    grid_spec = pltpu.PrefetchScalarGridSpec(
        num_scalar_prefetch=2,
        grid=(NP // BR,),
        in_specs=[
            pl.BlockSpec((BR, 4), lambda i, cs, ct: (i, 0)),
            pl.BlockSpec((BR, 1), lambda i, cs, ct: (i, 0)),
            pl.BlockSpec((NT, 8, 128), lambda i, cs, ct: (0, 0, 0)),
        ],
        out_specs=pl.BlockSpec((BR, K), lambda i, cs, ct: (i, 0)),
        scratch_shapes=[pltpu.VMEM((NT, BR, 128), jnp.int32)],
    )
    return pl.pallas_call(
        _knn_body,
        grid_spec=grid_spec,
        out_shape=jax.ShapeDtypeStruct((NP, K), jnp.int32),
    )(cs, ct, rows, rowsb, cols)


NW = 32
WT = NP // NW
CT_ = 4
CR = CT_ * K
NCH = WT // CT_


def _msg_call(h, ea, idxf):
    nk = ea.shape[0]
    mesh = plsc.VectorSubcoreMesh(core_axis_name="c", subcore_axis_name="s")

    @functools.partial(
        pl.kernel, mesh=mesh,
        out_type=jax.ShapeDtypeStruct((NP, H), jnp.float32),
        scratch_types=[
            pltpu.VMEM((WT * K,), jnp.int32),
            pltpu.VMEM((2, CR, H), jnp.float32),
            pltpu.VMEM((2, CR, H), jnp.float32),
            pltpu.VMEM((2, CT_, H), jnp.float32),
            pltpu.SemaphoreType.DMA,
            pltpu.SemaphoreType.DMA, pltpu.SemaphoreType.DMA,
            pltpu.SemaphoreType.DMA, pltpu.SemaphoreType.DMA,
            pltpu.SemaphoreType.DMA, pltpu.SemaphoreType.DMA,
        ],
    )
    def k(h_hbm, ea_hbm, idx_hbm, out_hbm, idxbuf, gbuf, ebuf, obuf,
          sem_i, sg0, sg1, se0, se1, so0, so1):
        wid = lax.axis_index("s") * 2 + lax.axis_index("c")
        base_t = wid * WT
        pltpu.async_copy(idx_hbm.at[pl.ds(base_t * K, WT * K)], idxbuf,
                         sem_i).wait()
        sgs = (sg0, sg1)
        ses = (se0, se1)
        sos = (so0, so1)

        def start(ch, b):
            pltpu.async_copy(
                h_hbm.at[idxbuf.at[pl.ds(ch * CR, CR)]], gbuf.at[b], sgs[b])
            base_e = jnp.minimum((base_t + ch * CT_) * K, nk - CR)
            pltpu.async_copy(ea_hbm.at[pl.ds(base_e, CR)], ebuf.at[b],
                             ses[b])

        start(0, 0)
        start(1, 1)

        def chunk(ch, b):
            pltpu.make_async_copy(h_hbm.at[pl.ds(0, CR)], gbuf.at[b],
                                  sgs[b]).wait()
            pltpu.make_async_copy(ea_hbm.at[pl.ds(0, CR)], ebuf.at[b],
                                  ses[b]).wait()
            for tloc in range(CT_):
                def jbody(j, accs):
                    row = tloc * K + j
                    new = []
                    for c in range(H // 16):
                        hv = gbuf[b, row, pl.ds(c * 16, 16)]
                        ev = ebuf[b, row, pl.ds(c * 16, 16)]
                        new.append(accs[c] + jnp.maximum(hv + ev, 0.0))
                    return tuple(new)

                accs = tuple(jnp.zeros((16,), jnp.float32)
                             for _ in range(H // 16))
                accs = lax.fori_loop(0, K, jbody, accs)
                for c in range(H // 16):
                    obuf[b, tloc, pl.ds(c * 16, 16)] = accs[c]
            pltpu.async_copy(
                obuf.at[b], out_hbm.at[pl.ds(base_t + ch * CT_, CT_)],
                sos[b])

        def loop(ch2, _):
            for b in range(2):
                ch = ch2 * 2 + b
                @pl.when(ch2 > 0)
                def _():
                    pltpu.make_async_copy(
                        obuf.at[b], out_hbm.at[pl.ds(0, CT_)], sos[b]).wait()
                chunk(ch, b)

                @pl.when(ch + 2 < NCH)
                def _():
                    start(ch + 2, b)
            return 0

        lax.fori_loop(0, NCH // 2, loop, 0)
        for b in range(2):
            pltpu.make_async_copy(obuf.at[b], out_hbm.at[pl.ds(0, CT_)],
                                  sos[b]).wait()

    return k(h, ea, idxf)


def _mlp2_body(x_ref, w1_ref, b1_ref, w2_ref, b2_ref, o_ref):
    h1 = jnp.maximum(
        jnp.dot(x_ref[...], w1_ref[...], preferred_element_type=jnp.float32)
        + b1_ref[...][None, :], 0.0)
    o_ref[...] = (
        jnp.dot(h1, w2_ref[...], preferred_element_type=jnp.float32)
        + b2_ref[...][None, :])


def _mlp2_call(x, w1, b1, w2, b2, bm):
    m, din = x.shape
    h = w1.shape[1]
    assert m % bm == 0
    return pl.pallas_call(
        _mlp2_body,
        grid=(m // bm,),
        in_specs=[
            pl.BlockSpec((bm, din), lambda i: (i, 0)),
            pl.BlockSpec((din, h), lambda i: (0, 0)),
            pl.BlockSpec((h,), lambda i: (0,)),
            pl.BlockSpec((h, h), lambda i: (0, 0)),
            pl.BlockSpec((h,), lambda i: (0,)),
        ],
        out_specs=pl.BlockSpec((bm, h), lambda i: (i, 0)),
        out_shape=jax.ShapeDtypeStruct((m, h), jnp.float32),
    )(x, w1, b1, w2, b2)


def _conv_body(agg_ref, h_ref, w1_ref, b1_ref, w2_ref, b2_ref,
               g_ref, bb_ref, o_ref):
    x = agg_ref[...] + h_ref[...]
    h1 = jnp.maximum(
        jnp.dot(x, w1_ref[...], preferred_element_type=jnp.float32)
        + b1_ref[...][None, :], 0.0)
    hn = (jnp.dot(h1, w2_ref[...], preferred_element_type=jnp.float32)
          + b2_ref[...][None, :])
    y = h_ref[...] + hn
    mu = jnp.mean(y, axis=-1, keepdims=True)
    c = y - mu
    var = jnp.mean(c * c, axis=-1, keepdims=True)
    o_ref[...] = c * jax.lax.rsqrt(var + 1e-5) * g_ref[...][None, :] \
        + bb_ref[...][None, :]


def _conv_call(agg, h, w1, b1, w2, b2, g, bb, bm):
    m = h.shape[0]
    assert m % bm == 0
    return pl.pallas_call(
        _conv_body,
        grid=(m // bm,),
        in_specs=[
            pl.BlockSpec((bm, H), lambda i: (i, 0)),
            pl.BlockSpec((bm, H), lambda i: (i, 0)),
            pl.BlockSpec((H, H), lambda i: (0, 0)),
            pl.BlockSpec((H,), lambda i: (0,)),
            pl.BlockSpec((H, H), lambda i: (0, 0)),
            pl.BlockSpec((H,), lambda i: (0,)),
            pl.BlockSpec((H,), lambda i: (0,)),
            pl.BlockSpec((H,), lambda i: (0,)),
        ],
        out_specs=pl.BlockSpec((bm, H), lambda i: (i, 0)),
        out_shape=jax.ShapeDtypeStruct((m, h.shape[1]), jnp.float32),
    )(agg, h, w1, b1, w2, b2, g, bb)


def kernel(cloud_x, cloud_batch, node_W1, node_b1, node_W2, node_b2,
           edge_W1, edge_b1, edge_W2, edge_b2, conv_W1, conv_b1, conv_W2,
           conv_b2, ln_g, ln_b):
    n = cloud_x.shape[0]
    xyz = cloud_x[:, :3]
    sq = jnp.sum(xyz * xyz, axis=1)
    batch = cloud_batch.astype(jnp.int32)

    gids = jnp.arange(NGRAPH, dtype=jnp.int32)
    starts = jnp.searchsorted(batch, gids, side="left").astype(jnp.int32)
    ends = jnp.searchsorted(batch, gids, side="right").astype(jnp.int32)

    nblk = NP // BR
    r0 = jnp.arange(nblk, dtype=jnp.int32) * BR
    r1 = jnp.minimum(r0 + BR - 1, n - 1)
    valid = r0 < n
    b0 = batch[jnp.minimum(r0, n - 1)]
    b1 = batch[r1]
    cs = jnp.where(valid, starts[b0] // 128, 0)
    ct = jnp.where(valid, (ends[b1] + 127) // 128 - cs, 1)

    rows = jnp.pad(jnp.concatenate([xyz, sq[:, None]], axis=1),
                   ((0, NP - n), (0, 0)))
    rowsb = jnp.pad(batch[:, None], ((0, NP - n), (0, 0)),
                    constant_values=-2)
    colsT = jnp.concatenate([
        jnp.pad(xyz.T, ((0, 0), (0, NP - n))),
        jnp.pad(sq[None], ((0, 0), (0, NP - n))),
        jnp.pad(batch[None].astype(jnp.float32), ((0, 0), (0, NP - n)),
                constant_values=-1.0),
        jnp.zeros((3, NP), jnp.float32),
    ]).reshape(8, NT, 128).transpose(1, 0, 2)

    idx = _knn_call(cs, ct, rows, rowsb, colsT)[:n]
    idxf = jnp.pad(idx, ((0, NP - n), (0, 0))).reshape(-1)

    xs = xyz[idx]
    delta = xyz[:, None, :] - xs
    dist = jnp.sqrt(jnp.sum(delta * delta, axis=-1, keepdims=True))
    raw_edge = jnp.concatenate(
        [delta, dist, jnp.zeros((n, K, 4), jnp.float32)], axis=-1)

    BM = 128

    ew1 = jnp.concatenate([edge_W1, jnp.zeros((4, H), jnp.float32)], axis=0)
    edge_attr = _mlp2_call(raw_edge.reshape(n * K, 8), ew1, edge_b1,
                           edge_W2, edge_b2, 640)

    xin = jnp.pad(cloud_x, ((0, NP - n), (0, 1)))
    nw1 = jnp.concatenate([node_W1, jnp.zeros((1, H), jnp.float32)], axis=0)
    h = _mlp2_call(xin, nw1, node_b1, node_W2, node_b2, 512)

    for i in range(L):
        agg = _msg_call(h, edge_attr, idxf)
        h = _conv_call(agg, h, conv_W1[i], conv_b1[i], conv_W2[i],
                       conv_b2[i], ln_g[i], ln_b[i], BM)
    return h[:n]


# speedup vs baseline: 1.2608x; 1.2608x over previous
import functools

import jax
import jax.numpy as jnp
from jax import lax
from jax.experimental import pallas as pl
from jax.experimental.pallas import tpu as pltpu
from jax.experimental.pallas import tpu_sc as plsc

N = 10000
K = 32
H = 128
L = 6
NGRAPH = 16

BR = 128
NP = 10240
NT = NP // 128


def _knn_body(cs_ref, ct_ref, rows_ref, rowsb_ref, cols_ref, idx_ref, kscr):
    i = pl.program_id(0)
    c_start = cs_ref[i]
    n_t = ct_ref[i]

    xr = rows_ref[:, 0:1]
    yr = rows_ref[:, 1:2]
    zr = rows_ref[:, 2:3]
    sqr = rows_ref[:, 3:4]
    rbf = rowsb_ref[...].astype(jnp.float32)
    rif = (i * BR + jax.lax.broadcasted_iota(jnp.int32, (BR, 1), 0)
           ).astype(jnp.float32)

    inf = jnp.float32(jnp.inf)

    def dist_tile(t, _):
        tt = c_start + t
        c = cols_ref[pl.ds(tt, 1)][0]
        xc, yc, zc, sqc, bc = c[0:1], c[1:2], c[2:3], c[3:4], c[4:5]
        dot = xr * xc + yr * yc + zr * zc
        d = sqr + sqc - 2.0 * dot
        colf = (tt * 128
                + jax.lax.broadcasted_iota(jnp.int32, (1, 128), 1)
                ).astype(jnp.float32)
        msk = (bc != rbf) | (colf == rif)
        d = jnp.where(msk, inf, d)
        b = d.view(jnp.int32)
        key = b ^ ((b >> 31) & jnp.int32(0x7FFFFFFF))
        kscr[pl.ds(t, 1)] = key[None]
        return 0

    jax.lax.fori_loop(0, n_t, dist_tile, 0)

    ones_col = jnp.ones((128, 1), jnp.float32)

    def bis(it, carry):
        lo, hi = carry
        mid = (lo >> 1) + (hi >> 1) + (lo & hi & 1)

        def cnt_tile(t, acc):
            k = kscr[pl.ds(t, 1)][0]
            return acc + jnp.where(k <= mid, 1.0, 0.0)

        accl = jax.lax.fori_loop(0, n_t, cnt_tile,
                                 jnp.zeros((BR, 128), jnp.float32))
        cnt = jnp.dot(accl, ones_col, preferred_element_type=jnp.float32)
        pick = cnt >= jnp.float32(K)
        return (jnp.where(pick, lo, mid + 1), jnp.where(pick, mid, hi))

    lo0 = jnp.full((BR, 1), jnp.int32(-2**31))
    hi0 = jnp.full((BR, 1), jnp.int32(2**31 - 1))
    lo, hi = jax.lax.fori_loop(0, 32, bis, (lo0, hi0))
    tstar = hi

    def cntlt_tile(t, acc):
        k = kscr[pl.ds(t, 1)][0]
        return acc + jnp.where(k < tstar, 1.0, 0.0)

    clt = jax.lax.fori_loop(0, n_t, cntlt_tile,
                            jnp.zeros((BR, 128), jnp.float32))
    quota = jnp.float32(K) - jnp.dot(clt, ones_col,
                                     preferred_element_type=jnp.float32)

    tri = (jax.lax.broadcasted_iota(jnp.int32, (128, 128), 0)
           <= jax.lax.broadcasted_iota(jnp.int32, (128, 128), 1)
           ).astype(jnp.float32)
    slot_iota = jax.lax.broadcasted_iota(jnp.int32, (1, K), 1)

    def ext_tile(t, carry):
        acc, ce, cc = carry
        tt = c_start + t
        k = kscr[pl.ds(t, 1)][0]
        m_eq_f = jnp.where(k == tstar, 1.0, 0.0)
        peq = jnp.dot(m_eq_f, tri, preferred_element_type=jnp.float32)
        chosen_f = jnp.where(k < tstar, 1.0,
                             m_eq_f * jnp.where(peq + ce <= quota, 1.0, 0.0))
        rank = jnp.dot(chosen_f, tri,
                       preferred_element_type=jnp.float32) + cc
        lanef = jax.lax.broadcasted_iota(jnp.int32, (BR, 128), 1
                                         ).astype(jnp.float32)
        rank_sel = rank * chosen_f
        base_f = jnp.float32(1.0) * (tt * 128)
        for s in range(K):
            m_s = jnp.where(rank_sel == jnp.float32(s + 1), 1.0, 0.0)
            cnt_s = jnp.dot(m_s, ones_col,
                            preferred_element_type=jnp.float32)
            lsum = jnp.dot(m_s * lanef, ones_col,
                           preferred_element_type=jnp.float32)
            contrib = base_f * cnt_s + lsum
            acc = acc + contrib * (slot_iota == s).astype(jnp.float32)
        ce = ce + jnp.dot(m_eq_f, ones_col,
                          preferred_element_type=jnp.float32)
        cc = cc + jnp.dot(chosen_f, ones_col,
                          preferred_element_type=jnp.float32)
        return acc, ce, cc

    acc0 = jnp.zeros((BR, K), jnp.float32)
    z = jnp.zeros((BR, 1), jnp.float32)
    acc, _, _ = jax.lax.fori_loop(0, n_t, ext_tile, (acc0, z, z))
    idx_ref[...] = acc.astype(jnp.int32)


def _knn_call(cs, ct, rows, rowsb, cols):
    grid_spec = pltpu.PrefetchScalarGridSpec(
        num_scalar_prefetch=2,
        grid=(NP // BR,),
        in_specs=[
            pl.BlockSpec((BR, 4), lambda i, cs, ct: (i, 0)),
            pl.BlockSpec((BR, 1), lambda i, cs, ct: (i, 0)),
            pl.BlockSpec((NT, 8, 128), lambda i, cs, ct: (0, 0, 0)),
        ],
        out_specs=pl.BlockSpec((BR, K), lambda i, cs, ct: (i, 0)),
        scratch_shapes=[pltpu.VMEM((NT, BR, 128), jnp.int32)],
    )
    return pl.pallas_call(
        _knn_body,
        grid_spec=grid_spec,
        out_shape=jax.ShapeDtypeStruct((NP, K), jnp.int32),
    )(cs, ct, rows, rowsb, cols)


NW = 32
WT = NP // NW
CT_ = 4
CR = CT_ * K
NCH = WT // CT_


def _msg_call(h, ea, idxf):
    nk = ea.shape[0]
    mesh = plsc.VectorSubcoreMesh(core_axis_name="c", subcore_axis_name="s")

    @functools.partial(
        pl.kernel, mesh=mesh,
        out_type=jax.ShapeDtypeStruct((NP, H), jnp.float32),
        scratch_types=[
            pltpu.VMEM((WT * K,), jnp.int32),
            pltpu.VMEM((2, CR, H), jnp.float32),
            pltpu.VMEM((2, CR, H), jnp.float32),
            pltpu.VMEM((2, CT_, H), jnp.float32),
            pltpu.SemaphoreType.DMA,
            pltpu.SemaphoreType.DMA, pltpu.SemaphoreType.DMA,
            pltpu.SemaphoreType.DMA, pltpu.SemaphoreType.DMA,
            pltpu.SemaphoreType.DMA, pltpu.SemaphoreType.DMA,
        ],
    )
    def k(h_hbm, ea_hbm, idx_hbm, out_hbm, idxbuf, gbuf, ebuf, obuf,
          sem_i, sg0, sg1, se0, se1, so0, so1):
        wid = lax.axis_index("s") * 2 + lax.axis_index("c")
        base_t = wid * WT
        pltpu.async_copy(idx_hbm.at[pl.ds(base_t * K, WT * K)], idxbuf,
                         sem_i).wait()
        sgs = (sg0, sg1)
        ses = (se0, se1)
        sos = (so0, so1)

        def start(ch, b):
            pltpu.async_copy(
                h_hbm.at[idxbuf.at[pl.ds(ch * CR, CR)]], gbuf.at[b], sgs[b])
            base_e = jnp.minimum((base_t + ch * CT_) * K, nk - CR)
            pltpu.async_copy(ea_hbm.at[pl.ds(base_e, CR)], ebuf.at[b],
                             ses[b])

        start(0, 0)
        start(1, 1)

        def chunk(ch, b):
            pltpu.make_async_copy(h_hbm.at[pl.ds(0, CR)], gbuf.at[b],
                                  sgs[b]).wait()
            pltpu.make_async_copy(ea_hbm.at[pl.ds(0, CR)], ebuf.at[b],
                                  ses[b]).wait()
            for tloc in range(CT_):
                def jbody(j, accs):
                    row = tloc * K + j
                    new = []
                    for c in range(H // 16):
                        hv = gbuf[b, row, pl.ds(c * 16, 16)]
                        ev = ebuf[b, row, pl.ds(c * 16, 16)]
                        new.append(accs[c] + jnp.maximum(hv + ev, 0.0))
                    return tuple(new)

                accs = tuple(jnp.zeros((16,), jnp.float32)
                             for _ in range(H // 16))
                accs = lax.fori_loop(0, K, jbody, accs)
                for c in range(H // 16):
                    obuf[b, tloc, pl.ds(c * 16, 16)] = accs[c]
            pltpu.async_copy(
                obuf.at[b], out_hbm.at[pl.ds(base_t + ch * CT_, CT_)],
                sos[b])

        def loop(ch2, _):
            for b in range(2):
                ch = ch2 * 2 + b
                @pl.when(ch2 > 0)
                def _():
                    pltpu.make_async_copy(
                        obuf.at[b], out_hbm.at[pl.ds(0, CT_)], sos[b]).wait()
                chunk(ch, b)

                @pl.when(ch + 2 < NCH)
                def _():
                    start(ch + 2, b)
            return 0

        lax.fori_loop(0, NCH // 2, loop, 0)
        for b in range(2):
            pltpu.make_async_copy(obuf.at[b], out_hbm.at[pl.ds(0, CT_)],
                                  sos[b]).wait()

    return k(h, ea, idxf)


def _mlp2_body(x_ref, w1_ref, b1_ref, w2_ref, b2_ref, o_ref):
    h1 = jnp.maximum(
        jnp.dot(x_ref[...], w1_ref[...], preferred_element_type=jnp.float32)
        + b1_ref[...][None, :], 0.0)
    o_ref[...] = (
        jnp.dot(h1, w2_ref[...], preferred_element_type=jnp.float32)
        + b2_ref[...][None, :])


def _mlp2_call(x, w1, b1, w2, b2, bm):
    m, din = x.shape
    h = w1.shape[1]
    assert m % bm == 0
    return pl.pallas_call(
        _mlp2_body,
        grid=(m // bm,),
        in_specs=[
            pl.BlockSpec((bm, din), lambda i: (i, 0)),
            pl.BlockSpec((din, h), lambda i: (0, 0)),
            pl.BlockSpec((h,), lambda i: (0,)),
            pl.BlockSpec((h, h), lambda i: (0, 0)),
            pl.BlockSpec((h,), lambda i: (0,)),
        ],
        out_specs=pl.BlockSpec((bm, h), lambda i: (i, 0)),
        out_shape=jax.ShapeDtypeStruct((m, h), jnp.float32),
    )(x, w1, b1, w2, b2)


def _conv_body(agg_ref, h_ref, w1_ref, b1_ref, w2_ref, b2_ref,
               g_ref, bb_ref, o_ref):
    x = agg_ref[...] + h_ref[...]
    h1 = jnp.maximum(
        jnp.dot(x, w1_ref[...], preferred_element_type=jnp.float32)
        + b1_ref[...][None, :], 0.0)
    hn = (jnp.dot(h1, w2_ref[...], preferred_element_type=jnp.float32)
          + b2_ref[...][None, :])
    y = h_ref[...] + hn
    mu = jnp.mean(y, axis=-1, keepdims=True)
    c = y - mu
    var = jnp.mean(c * c, axis=-1, keepdims=True)
    o_ref[...] = c * jax.lax.rsqrt(var + 1e-5) * g_ref[...][None, :] \
        + bb_ref[...][None, :]


def _conv_call(agg, h, w1, b1, w2, b2, g, bb, bm):
    m = h.shape[0]
    assert m % bm == 0
    return pl.pallas_call(
        _conv_body,
        grid=(m // bm,),
        in_specs=[
            pl.BlockSpec((bm, H), lambda i: (i, 0)),
            pl.BlockSpec((bm, H), lambda i: (i, 0)),
            pl.BlockSpec((H, H), lambda i: (0, 0)),
            pl.BlockSpec((H,), lambda i: (0,)),
            pl.BlockSpec((H, H), lambda i: (0, 0)),
            pl.BlockSpec((H,), lambda i: (0,)),
            pl.BlockSpec((H,), lambda i: (0,)),
            pl.BlockSpec((H,), lambda i: (0,)),
        ],
        out_specs=pl.BlockSpec((bm, H), lambda i: (i, 0)),
        out_shape=jax.ShapeDtypeStruct((m, h.shape[1]), jnp.float32),
    )(agg, h, w1, b1, w2, b2, g, bb)


def kernel(cloud_x, cloud_batch, node_W1, node_b1, node_W2, node_b2,
           edge_W1, edge_b1, edge_W2, edge_b2, conv_W1, conv_b1, conv_W2,
           conv_b2, ln_g, ln_b):
    n = cloud_x.shape[0]
    xyz = cloud_x[:, :3]
    sq = jnp.sum(xyz * xyz, axis=1)
    batch = cloud_batch.astype(jnp.int32)

    gids = jnp.arange(NGRAPH, dtype=jnp.int32)
    starts = jnp.searchsorted(batch, gids, side="left").astype(jnp.int32)
    ends = jnp.searchsorted(batch, gids, side="right").astype(jnp.int32)

    nblk = NP // BR
    r0 = jnp.arange(nblk, dtype=jnp.int32) * BR
    r1 = jnp.minimum(r0 + BR - 1, n - 1)
    valid = r0 < n
    b0 = batch[jnp.minimum(r0, n - 1)]
    b1 = batch[r1]
    cs = jnp.where(valid, starts[b0] // 128, 0)
    ct = jnp.where(valid, (ends[b1] + 127) // 128 - cs, 1)

    rows = jnp.pad(jnp.concatenate([xyz, sq[:, None]], axis=1),
                   ((0, NP - n), (0, 0)))
    rowsb = jnp.pad(batch[:, None], ((0, NP - n), (0, 0)),
                    constant_values=-2)
    colsT = jnp.concatenate([
        jnp.pad(xyz.T, ((0, 0), (0, NP - n))),
        jnp.pad(sq[None], ((0, 0), (0, NP - n))),
        jnp.pad(batch[None].astype(jnp.float32), ((0, 0), (0, NP - n)),
                constant_values=-1.0),
        jnp.zeros((3, NP), jnp.float32),
    ]).reshape(8, NT, 128).transpose(1, 0, 2)

    idx = _knn_call(cs, ct, rows, rowsb, colsT)[:n]
    idxf = jnp.pad(idx, ((0, NP - n), (0, 0))).reshape(-1)

    xs = xyz[idx]
    delta = xyz[:, None, :] - xs
    dist = jnp.sqrt(jnp.sum(delta * delta, axis=-1, keepdims=True))
    raw_edge = jnp.concatenate(
        [delta, dist, jnp.zeros((n, K, 4), jnp.float32)], axis=-1)

    BM = 128

    ew1 = jnp.concatenate([edge_W1, jnp.zeros((4, H), jnp.float32)], axis=0)
    edge_attr = _mlp2_call(raw_edge.reshape(n * K, 8), ew1, edge_b1,
                           edge_W2, edge_b2, 640)

    xin = jnp.pad(cloud_x, ((0, NP - n), (0, 1)))
    nw1 = jnp.concatenate([node_W1, jnp.zeros((1, H), jnp.float32)], axis=0)
    h = _mlp2_call(xin, nw1, node_b1, node_W2, node_b2, 512)

    for i in range(L):
        agg = _msg_call(h, edge_attr, idxf)
        h = _conv_call(agg, h, conv_W1[i], conv_b1[i], conv_W2[i],
                       conv_b2[i], ln_g[i], ln_b[i], BM)
    return h[:n]


# speedup vs baseline: 1.4208x; 1.1269x over previous
import functools

import jax
import jax.numpy as jnp
from jax import lax
from jax.experimental import pallas as pl
from jax.experimental.pallas import tpu as pltpu
from jax.experimental.pallas import tpu_sc as plsc

N = 10000
K = 32
H = 128
L = 6
NGRAPH = 16

BR = 256
NP = 10240
NT = NP // 128


def _knn_body(cs_ref, ct_ref, rows_ref, rowsb_ref, cols_ref, idx_ref, kscr):
    i = pl.program_id(0)
    c_start = cs_ref[i]
    n_t = ct_ref[i]

    xr = rows_ref[:, 0:1]
    yr = rows_ref[:, 1:2]
    zr = rows_ref[:, 2:3]
    sqr = rows_ref[:, 3:4]
    rbf = rowsb_ref[...].astype(jnp.float32)
    rif = (i * BR + jax.lax.broadcasted_iota(jnp.int32, (BR, 1), 0)
           ).astype(jnp.float32)

    inf = jnp.float32(jnp.inf)

    def dist_tile(t, _):
        tt = c_start + t
        c = cols_ref[pl.ds(tt, 1)][0]
        xc, yc, zc, sqc, bc = c[0:1], c[1:2], c[2:3], c[3:4], c[4:5]
        dot = xr * xc + yr * yc + zr * zc
        d = sqr + sqc - 2.0 * dot
        colf = (tt * 128
                + jax.lax.broadcasted_iota(jnp.int32, (1, 128), 1)
                ).astype(jnp.float32)
        msk = (bc != rbf) | (colf == rif)
        d = jnp.where(msk, inf, d)
        b = d.view(jnp.int32)
        key = b ^ ((b >> 31) & jnp.int32(0x7FFFFFFF))
        kscr[pl.ds(t, 1)] = key[None]
        return 0

    jax.lax.fori_loop(0, n_t, dist_tile, 0)

    ones_col = jnp.ones((128, 1), jnp.float32)

    def bis(it, carry):
        lo, hi = carry
        mid = (lo >> 1) + (hi >> 1) + (lo & hi & 1)

        def cnt_tile(t, acc):
            k = kscr[pl.ds(t, 1)][0]
            return acc + jnp.where(k <= mid, 1.0, 0.0)

        accl = jax.lax.fori_loop(0, n_t, cnt_tile,
                                 jnp.zeros((BR, 128), jnp.float32))
        cnt = jnp.dot(accl, ones_col, preferred_element_type=jnp.float32)
        pick = cnt >= jnp.float32(K)
        return (jnp.where(pick, lo, mid + 1), jnp.where(pick, mid, hi))

    lo0 = jnp.full((BR, 1), jnp.int32(-2**31))
    hi0 = jnp.full((BR, 1), jnp.int32(2**31 - 1))
    lo, hi = jax.lax.fori_loop(0, 32, bis, (lo0, hi0))
    tstar = hi

    def cntlt_tile(t, acc):
        k = kscr[pl.ds(t, 1)][0]
        return acc + jnp.where(k < tstar, 1.0, 0.0)

    clt = jax.lax.fori_loop(0, n_t, cntlt_tile,
                            jnp.zeros((BR, 128), jnp.float32))
    quota = jnp.float32(K) - jnp.dot(clt, ones_col,
                                     preferred_element_type=jnp.float32)

    tri = (jax.lax.broadcasted_iota(jnp.int32, (128, 128), 0)
           <= jax.lax.broadcasted_iota(jnp.int32, (128, 128), 1)
           ).astype(jnp.float32)
    slot_iota = jax.lax.broadcasted_iota(jnp.int32, (1, K), 1)

    def ext_tile(t, carry):
        acc, ce, cc = carry
        tt = c_start + t
        k = kscr[pl.ds(t, 1)][0]
        m_eq_f = jnp.where(k == tstar, 1.0, 0.0)
        peq = jnp.dot(m_eq_f, tri, preferred_element_type=jnp.float32)
        chosen_f = jnp.where(k < tstar, 1.0,
                             m_eq_f * jnp.where(peq + ce <= quota, 1.0, 0.0))
        rank = jnp.dot(chosen_f, tri,
                       preferred_element_type=jnp.float32) + cc
        lanef = jax.lax.broadcasted_iota(jnp.int32, (BR, 128), 1
                                         ).astype(jnp.float32)
        rank_sel = rank * chosen_f
        base_f = jnp.float32(1.0) * (tt * 128)
        for s in range(K):
            m_s = jnp.where(rank_sel == jnp.float32(s + 1), 1.0, 0.0)
            cnt_s = jnp.dot(m_s, ones_col,
                            preferred_element_type=jnp.float32)
            lsum = jnp.dot(m_s * lanef, ones_col,
                           preferred_element_type=jnp.float32)
            contrib = base_f * cnt_s + lsum
            acc = acc + contrib * (slot_iota == s).astype(jnp.float32)
        ce = ce + jnp.dot(m_eq_f, ones_col,
                          preferred_element_type=jnp.float32)
        cc = cc + jnp.dot(chosen_f, ones_col,
                          preferred_element_type=jnp.float32)
        return acc, ce, cc

    acc0 = jnp.zeros((BR, K), jnp.float32)
    z = jnp.zeros((BR, 1), jnp.float32)
    acc, _, _ = jax.lax.fori_loop(0, n_t, ext_tile, (acc0, z, z))
    idx_ref[...] = acc.astype(jnp.int32)


def _knn_call(cs, ct, rows, rowsb, cols):
    grid_spec = pltpu.PrefetchScalarGridSpec(
        num_scalar_prefetch=2,
        grid=(NP // BR,),
        in_specs=[
            pl.BlockSpec((BR, 4), lambda i, cs, ct: (i, 0)),
            pl.BlockSpec((BR, 1), lambda i, cs, ct: (i, 0)),
            pl.BlockSpec((NT, 8, 128), lambda i, cs, ct: (0, 0, 0)),
        ],
        out_specs=pl.BlockSpec((BR, K), lambda i, cs, ct: (i, 0)),
        scratch_shapes=[pltpu.VMEM((NT, BR, 128), jnp.int32)],
    )
    return pl.pallas_call(
        _knn_body,
        grid_spec=grid_spec,
        out_shape=jax.ShapeDtypeStruct((NP, K), jnp.int32),
    )(cs, ct, rows, rowsb, cols)


NW = 32
WT = NP // NW
CT_ = 4
CR = CT_ * K
NCH = WT // CT_


def _msg_call(h, ea, idxf):
    nk = ea.shape[0]
    mesh = plsc.VectorSubcoreMesh(core_axis_name="c", subcore_axis_name="s")

    @functools.partial(
        pl.kernel, mesh=mesh,
        out_type=jax.ShapeDtypeStruct((NP, H), jnp.float32),
        scratch_types=[
            pltpu.VMEM((WT * K,), jnp.int32),
            pltpu.VMEM((2, CR, H), jnp.float32),
            pltpu.VMEM((2, CR, H), jnp.float32),
            pltpu.VMEM((2, CT_, H), jnp.float32),
            pltpu.SemaphoreType.DMA,
            pltpu.SemaphoreType.DMA, pltpu.SemaphoreType.DMA,
            pltpu.SemaphoreType.DMA, pltpu.SemaphoreType.DMA,
            pltpu.SemaphoreType.DMA, pltpu.SemaphoreType.DMA,
        ],
    )
    def k(h_hbm, ea_hbm, idx_hbm, out_hbm, idxbuf, gbuf, ebuf, obuf,
          sem_i, sg0, sg1, se0, se1, so0, so1):
        wid = lax.axis_index("s") * 2 + lax.axis_index("c")
        base_t = wid * WT
        pltpu.async_copy(idx_hbm.at[pl.ds(base_t * K, WT * K)], idxbuf,
                         sem_i).wait()
        sgs = (sg0, sg1)
        ses = (se0, se1)
        sos = (so0, so1)

        def start(ch, b):
            pltpu.async_copy(
                h_hbm.at[idxbuf.at[pl.ds(ch * CR, CR)]], gbuf.at[b], sgs[b])
            base_e = jnp.minimum((base_t + ch * CT_) * K, nk - CR)
            pltpu.async_copy(ea_hbm.at[pl.ds(base_e, CR)], ebuf.at[b],
                             ses[b])

        start(0, 0)
        start(1, 1)

        def chunk(ch, b):
            pltpu.make_async_copy(h_hbm.at[pl.ds(0, CR)], gbuf.at[b],
                                  sgs[b]).wait()
            pltpu.make_async_copy(ea_hbm.at[pl.ds(0, CR)], ebuf.at[b],
                                  ses[b]).wait()
            for tloc in range(CT_):
                def jbody(j, accs):
                    row = tloc * K + j
                    new = []
                    for c in range(H // 16):
                        hv = gbuf[b, row, pl.ds(c * 16, 16)]
                        ev = ebuf[b, row, pl.ds(c * 16, 16)]
                        new.append(accs[c] + jnp.maximum(hv + ev, 0.0))
                    return tuple(new)

                accs = tuple(jnp.zeros((16,), jnp.float32)
                             for _ in range(H // 16))
                accs = lax.fori_loop(0, K, jbody, accs)
                for c in range(H // 16):
                    obuf[b, tloc, pl.ds(c * 16, 16)] = accs[c]
            pltpu.async_copy(
                obuf.at[b], out_hbm.at[pl.ds(base_t + ch * CT_, CT_)],
                sos[b])

        def loop(ch2, _):
            for b in range(2):
                ch = ch2 * 2 + b
                @pl.when(ch2 > 0)
                def _():
                    pltpu.make_async_copy(
                        obuf.at[b], out_hbm.at[pl.ds(0, CT_)], sos[b]).wait()
                chunk(ch, b)

                @pl.when(ch + 2 < NCH)
                def _():
                    start(ch + 2, b)
            return 0

        lax.fori_loop(0, NCH // 2, loop, 0)
        for b in range(2):
            pltpu.make_async_copy(obuf.at[b], out_hbm.at[pl.ds(0, CT_)],
                                  sos[b]).wait()

    return k(h, ea, idxf)


def _mlp2_body(x_ref, w1_ref, b1_ref, w2_ref, b2_ref, o_ref):
    h1 = jnp.maximum(
        jnp.dot(x_ref[...], w1_ref[...], preferred_element_type=jnp.float32)
        + b1_ref[...][None, :], 0.0)
    o_ref[...] = (
        jnp.dot(h1, w2_ref[...], preferred_element_type=jnp.float32)
        + b2_ref[...][None, :])


def _mlp2_call(x, w1, b1, w2, b2, bm):
    m, din = x.shape
    h = w1.shape[1]
    assert m % bm == 0
    return pl.pallas_call(
        _mlp2_body,
        grid=(m // bm,),
        in_specs=[
            pl.BlockSpec((bm, din), lambda i: (i, 0)),
            pl.BlockSpec((din, h), lambda i: (0, 0)),
            pl.BlockSpec((h,), lambda i: (0,)),
            pl.BlockSpec((h, h), lambda i: (0, 0)),
            pl.BlockSpec((h,), lambda i: (0,)),
        ],
        out_specs=pl.BlockSpec((bm, h), lambda i: (i, 0)),
        out_shape=jax.ShapeDtypeStruct((m, h), jnp.float32),
    )(x, w1, b1, w2, b2)


def _conv_body(agg_ref, h_ref, w1_ref, b1_ref, w2_ref, b2_ref,
               g_ref, bb_ref, o_ref):
    x = agg_ref[...] + h_ref[...]
    h1 = jnp.maximum(
        jnp.dot(x, w1_ref[...], preferred_element_type=jnp.float32)
        + b1_ref[...][None, :], 0.0)
    hn = (jnp.dot(h1, w2_ref[...], preferred_element_type=jnp.float32)
          + b2_ref[...][None, :])
    y = h_ref[...] + hn
    mu = jnp.mean(y, axis=-1, keepdims=True)
    c = y - mu
    var = jnp.mean(c * c, axis=-1, keepdims=True)
    o_ref[...] = c * jax.lax.rsqrt(var + 1e-5) * g_ref[...][None, :] \
        + bb_ref[...][None, :]


def _conv_call(agg, h, w1, b1, w2, b2, g, bb, bm):
    m = h.shape[0]
    assert m % bm == 0
    return pl.pallas_call(
        _conv_body,
        grid=(m // bm,),
        in_specs=[
            pl.BlockSpec((bm, H), lambda i: (i, 0)),
            pl.BlockSpec((bm, H), lambda i: (i, 0)),
            pl.BlockSpec((H, H), lambda i: (0, 0)),
            pl.BlockSpec((H,), lambda i: (0,)),
            pl.BlockSpec((H, H), lambda i: (0, 0)),
            pl.BlockSpec((H,), lambda i: (0,)),
            pl.BlockSpec((H,), lambda i: (0,)),
            pl.BlockSpec((H,), lambda i: (0,)),
        ],
        out_specs=pl.BlockSpec((bm, H), lambda i: (i, 0)),
        out_shape=jax.ShapeDtypeStruct((m, h.shape[1]), jnp.float32),
    )(agg, h, w1, b1, w2, b2, g, bb)


def kernel(cloud_x, cloud_batch, node_W1, node_b1, node_W2, node_b2,
           edge_W1, edge_b1, edge_W2, edge_b2, conv_W1, conv_b1, conv_W2,
           conv_b2, ln_g, ln_b):
    n = cloud_x.shape[0]
    xyz = cloud_x[:, :3]
    sq = jnp.sum(xyz * xyz, axis=1)
    batch = cloud_batch.astype(jnp.int32)

    gids = jnp.arange(NGRAPH, dtype=jnp.int32)
    starts = jnp.searchsorted(batch, gids, side="left").astype(jnp.int32)
    ends = jnp.searchsorted(batch, gids, side="right").astype(jnp.int32)

    nblk = NP // BR
    r0 = jnp.arange(nblk, dtype=jnp.int32) * BR
    r1 = jnp.minimum(r0 + BR - 1, n - 1)
    valid = r0 < n
    b0 = batch[jnp.minimum(r0, n - 1)]
    b1 = batch[r1]
    cs = jnp.where(valid, starts[b0] // 128, 0)
    ct = jnp.where(valid, (ends[b1] + 127) // 128 - cs, 1)

    rows = jnp.pad(jnp.concatenate([xyz, sq[:, None]], axis=1),
                   ((0, NP - n), (0, 0)))
    rowsb = jnp.pad(batch[:, None], ((0, NP - n), (0, 0)),
                    constant_values=-2)
    colsT = jnp.concatenate([
        jnp.pad(xyz.T, ((0, 0), (0, NP - n))),
        jnp.pad(sq[None], ((0, 0), (0, NP - n))),
        jnp.pad(batch[None].astype(jnp.float32), ((0, 0), (0, NP - n)),
                constant_values=-1.0),
        jnp.zeros((3, NP), jnp.float32),
    ]).reshape(8, NT, 128).transpose(1, 0, 2)

    idx = _knn_call(cs, ct, rows, rowsb, colsT)[:n]
    idxf = jnp.pad(idx, ((0, NP - n), (0, 0))).reshape(-1)

    xs = xyz[idx]
    delta = xyz[:, None, :] - xs
    dist = jnp.sqrt(jnp.sum(delta * delta, axis=-1, keepdims=True))
    raw_edge = jnp.concatenate(
        [delta, dist, jnp.zeros((n, K, 4), jnp.float32)], axis=-1)

    BM = 128

    ew1 = jnp.concatenate([edge_W1, jnp.zeros((4, H), jnp.float32)], axis=0)
    edge_attr = _mlp2_call(raw_edge.reshape(n * K, 8), ew1, edge_b1,
                           edge_W2, edge_b2, 640)

    xin = jnp.pad(cloud_x, ((0, NP - n), (0, 1)))
    nw1 = jnp.concatenate([node_W1, jnp.zeros((1, H), jnp.float32)], axis=0)
    h = _mlp2_call(xin, nw1, node_b1, node_W2, node_b2, 512)

    for i in range(L):
        agg = _msg_call(h, edge_attr, idxf)
        h = _conv_call(agg, h, conv_W1[i], conv_b1[i], conv_W2[i],
                       conv_b2[i], ln_g[i], ln_b[i], BM)
    return h[:n]


# speedup vs baseline: 1.4542x; 1.0235x over previous
import functools

import jax
import jax.numpy as jnp
from jax import lax
from jax.experimental import pallas as pl
from jax.experimental.pallas import tpu as pltpu
from jax.experimental.pallas import tpu_sc as plsc

N = 10000
K = 32
H = 128
L = 6
NGRAPH = 16

BR = 512
NP = 10240
NT = NP // 128


def _knn_body(cs_ref, ct_ref, rows_ref, rowsb_ref, cols_ref, idx_ref, kscr):
    i = pl.program_id(0)
    c_start = cs_ref[i]
    n_t = ct_ref[i]

    xr = rows_ref[:, 0:1]
    yr = rows_ref[:, 1:2]
    zr = rows_ref[:, 2:3]
    sqr = rows_ref[:, 3:4]
    rbf = rowsb_ref[...].astype(jnp.float32)
    rif = (i * BR + jax.lax.broadcasted_iota(jnp.int32, (BR, 1), 0)
           ).astype(jnp.float32)

    inf = jnp.float32(jnp.inf)

    def dist_tile(t, _):
        tt = c_start + t
        c = cols_ref[pl.ds(tt, 1)][0]
        xc, yc, zc, sqc, bc = c[0:1], c[1:2], c[2:3], c[3:4], c[4:5]
        dot = xr * xc + yr * yc + zr * zc
        d = sqr + sqc - 2.0 * dot
        colf = (tt * 128
                + jax.lax.broadcasted_iota(jnp.int32, (1, 128), 1)
                ).astype(jnp.float32)
        msk = (bc != rbf) | (colf == rif)
        d = jnp.where(msk, inf, d)
        b = d.view(jnp.int32)
        key = b ^ ((b >> 31) & jnp.int32(0x7FFFFFFF))
        kscr[pl.ds(t, 1)] = key[None]
        return 0

    jax.lax.fori_loop(0, n_t, dist_tile, 0)

    ones_col = jnp.ones((128, 1), jnp.float32)

    def bis(it, carry):
        lo, hi = carry
        mid = (lo >> 1) + (hi >> 1) + (lo & hi & 1)

        def cnt_tile(t, acc):
            k = kscr[pl.ds(t, 1)][0]
            return acc + jnp.where(k <= mid, 1.0, 0.0)

        accl = jax.lax.fori_loop(0, n_t, cnt_tile,
                                 jnp.zeros((BR, 128), jnp.float32))
        cnt = jnp.dot(accl, ones_col, preferred_element_type=jnp.float32)
        pick = cnt >= jnp.float32(K)
        return (jnp.where(pick, lo, mid + 1), jnp.where(pick, mid, hi))

    lo0 = jnp.full((BR, 1), jnp.int32(-2**31))
    hi0 = jnp.full((BR, 1), jnp.int32(2**31 - 1))
    lo, hi = jax.lax.fori_loop(0, 32, bis, (lo0, hi0))
    tstar = hi

    def cntlt_tile(t, acc):
        k = kscr[pl.ds(t, 1)][0]
        return acc + jnp.where(k < tstar, 1.0, 0.0)

    clt = jax.lax.fori_loop(0, n_t, cntlt_tile,
                            jnp.zeros((BR, 128), jnp.float32))
    quota = jnp.float32(K) - jnp.dot(clt, ones_col,
                                     preferred_element_type=jnp.float32)

    tri = (jax.lax.broadcasted_iota(jnp.int32, (128, 128), 0)
           <= jax.lax.broadcasted_iota(jnp.int32, (128, 128), 1)
           ).astype(jnp.float32)
    slot_iota = jax.lax.broadcasted_iota(jnp.int32, (1, K), 1)

    def ext_tile(t, carry):
        acc, ce, cc = carry
        tt = c_start + t
        k = kscr[pl.ds(t, 1)][0]
        m_eq_f = jnp.where(k == tstar, 1.0, 0.0)
        peq = jnp.dot(m_eq_f, tri, preferred_element_type=jnp.float32)
        chosen_f = jnp.where(k < tstar, 1.0,
                             m_eq_f * jnp.where(peq + ce <= quota, 1.0, 0.0))
        rank = jnp.dot(chosen_f, tri,
                       preferred_element_type=jnp.float32) + cc
        lanef = jax.lax.broadcasted_iota(jnp.int32, (BR, 128), 1
                                         ).astype(jnp.float32)
        rank_sel = rank * chosen_f
        base_f = jnp.float32(1.0) * (tt * 128)
        for s in range(K):
            m_s = jnp.where(rank_sel == jnp.float32(s + 1), 1.0, 0.0)
            cnt_s = jnp.dot(m_s, ones_col,
                            preferred_element_type=jnp.float32)
            lsum = jnp.dot(m_s * lanef, ones_col,
                           preferred_element_type=jnp.float32)
            contrib = base_f * cnt_s + lsum
            acc = acc + contrib * (slot_iota == s).astype(jnp.float32)
        ce = ce + jnp.dot(m_eq_f, ones_col,
                          preferred_element_type=jnp.float32)
        cc = cc + jnp.dot(chosen_f, ones_col,
                          preferred_element_type=jnp.float32)
        return acc, ce, cc

    acc0 = jnp.zeros((BR, K), jnp.float32)
    z = jnp.zeros((BR, 1), jnp.float32)
    acc, _, _ = jax.lax.fori_loop(0, n_t, ext_tile, (acc0, z, z))
    idx_ref[...] = acc.astype(jnp.int32)


def _knn_call(cs, ct, rows, rowsb, cols):
    grid_spec = pltpu.PrefetchScalarGridSpec(
        num_scalar_prefetch=2,
        grid=(NP // BR,),
        in_specs=[
            pl.BlockSpec((BR, 4), lambda i, cs, ct: (i, 0)),
            pl.BlockSpec((BR, 1), lambda i, cs, ct: (i, 0)),
            pl.BlockSpec((NT, 8, 128), lambda i, cs, ct: (0, 0, 0)),
        ],
        out_specs=pl.BlockSpec((BR, K), lambda i, cs, ct: (i, 0)),
        scratch_shapes=[pltpu.VMEM((NT, BR, 128), jnp.int32)],
    )
    return pl.pallas_call(
        _knn_body,
        grid_spec=grid_spec,
        out_shape=jax.ShapeDtypeStruct((NP, K), jnp.int32),
    )(cs, ct, rows, rowsb, cols)


NW = 32
WT = NP // NW
CT_ = 4
CR = CT_ * K
NCH = WT // CT_


def _msg_call(h, ea, idxf):
    nk = ea.shape[0]
    mesh = plsc.VectorSubcoreMesh(core_axis_name="c", subcore_axis_name="s")

    @functools.partial(
        pl.kernel, mesh=mesh,
        out_type=jax.ShapeDtypeStruct((NP, H), jnp.float32),
        scratch_types=[
            pltpu.VMEM((WT * K,), jnp.int32),
            pltpu.VMEM((2, CR, H), jnp.float32),
            pltpu.VMEM((2, CR, H), jnp.float32),
            pltpu.VMEM((2, CT_, H), jnp.float32),
            pltpu.SemaphoreType.DMA,
            pltpu.SemaphoreType.DMA, pltpu.SemaphoreType.DMA,
            pltpu.SemaphoreType.DMA, pltpu.SemaphoreType.DMA,
            pltpu.SemaphoreType.DMA, pltpu.SemaphoreType.DMA,
        ],
    )
    def k(h_hbm, ea_hbm, idx_hbm, out_hbm, idxbuf, gbuf, ebuf, obuf,
          sem_i, sg0, sg1, se0, se1, so0, so1):
        wid = lax.axis_index("s") * 2 + lax.axis_index("c")
        base_t = wid * WT
        pltpu.async_copy(idx_hbm.at[pl.ds(base_t * K, WT * K)], idxbuf,
                         sem_i).wait()
        sgs = (sg0, sg1)
        ses = (se0, se1)
        sos = (so0, so1)

        def start(ch, b):
            pltpu.async_copy(
                h_hbm.at[idxbuf.at[pl.ds(ch * CR, CR)]], gbuf.at[b], sgs[b])
            base_e = jnp.minimum((base_t + ch * CT_) * K, nk - CR)
            pltpu.async_copy(ea_hbm.at[pl.ds(base_e, CR)], ebuf.at[b],
                             ses[b])

        start(0, 0)
        start(1, 1)

        def chunk(ch, b):
            pltpu.make_async_copy(h_hbm.at[pl.ds(0, CR)], gbuf.at[b],
                                  sgs[b]).wait()
            pltpu.make_async_copy(ea_hbm.at[pl.ds(0, CR)], ebuf.at[b],
                                  ses[b]).wait()
            for tloc in range(CT_):
                def jbody(j, accs):
                    row = tloc * K + j
                    new = []
                    for c in range(H // 16):
                        hv = gbuf[b, row, pl.ds(c * 16, 16)]
                        ev = ebuf[b, row, pl.ds(c * 16, 16)]
                        new.append(accs[c] + jnp.maximum(hv + ev, 0.0))
                    return tuple(new)

                accs = tuple(jnp.zeros((16,), jnp.float32)
                             for _ in range(H // 16))
                accs = lax.fori_loop(0, K, jbody, accs)
                for c in range(H // 16):
                    obuf[b, tloc, pl.ds(c * 16, 16)] = accs[c]
            pltpu.async_copy(
                obuf.at[b], out_hbm.at[pl.ds(base_t + ch * CT_, CT_)],
                sos[b])

        def loop(ch2, _):
            for b in range(2):
                ch = ch2 * 2 + b
                @pl.when(ch2 > 0)
                def _():
                    pltpu.make_async_copy(
                        obuf.at[b], out_hbm.at[pl.ds(0, CT_)], sos[b]).wait()
                chunk(ch, b)

                @pl.when(ch + 2 < NCH)
                def _():
                    start(ch + 2, b)
            return 0

        lax.fori_loop(0, NCH // 2, loop, 0)
        for b in range(2):
            pltpu.make_async_copy(obuf.at[b], out_hbm.at[pl.ds(0, CT_)],
                                  sos[b]).wait()

    return k(h, ea, idxf)


def _mlp2_body(x_ref, w1_ref, b1_ref, w2_ref, b2_ref, o_ref):
    h1 = jnp.maximum(
        jnp.dot(x_ref[...], w1_ref[...], preferred_element_type=jnp.float32)
        + b1_ref[...][None, :], 0.0)
    o_ref[...] = (
        jnp.dot(h1, w2_ref[...], preferred_element_type=jnp.float32)
        + b2_ref[...][None, :])


def _mlp2_call(x, w1, b1, w2, b2, bm):
    m, din = x.shape
    h = w1.shape[1]
    assert m % bm == 0
    return pl.pallas_call(
        _mlp2_body,
        grid=(m // bm,),
        in_specs=[
            pl.BlockSpec((bm, din), lambda i: (i, 0)),
            pl.BlockSpec((din, h), lambda i: (0, 0)),
            pl.BlockSpec((h,), lambda i: (0,)),
            pl.BlockSpec((h, h), lambda i: (0, 0)),
            pl.BlockSpec((h,), lambda i: (0,)),
        ],
        out_specs=pl.BlockSpec((bm, h), lambda i: (i, 0)),
        out_shape=jax.ShapeDtypeStruct((m, h), jnp.float32),
    )(x, w1, b1, w2, b2)


def _conv_body(agg_ref, h_ref, w1_ref, b1_ref, w2_ref, b2_ref,
               g_ref, bb_ref, o_ref):
    x = agg_ref[...] + h_ref[...]
    h1 = jnp.maximum(
        jnp.dot(x, w1_ref[...], preferred_element_type=jnp.float32)
        + b1_ref[...][None, :], 0.0)
    hn = (jnp.dot(h1, w2_ref[...], preferred_element_type=jnp.float32)
          + b2_ref[...][None, :])
    y = h_ref[...] + hn
    mu = jnp.mean(y, axis=-1, keepdims=True)
    c = y - mu
    var = jnp.mean(c * c, axis=-1, keepdims=True)
    o_ref[...] = c * jax.lax.rsqrt(var + 1e-5) * g_ref[...][None, :] \
        + bb_ref[...][None, :]


def _conv_call(agg, h, w1, b1, w2, b2, g, bb, bm):
    m = h.shape[0]
    assert m % bm == 0
    return pl.pallas_call(
        _conv_body,
        grid=(m // bm,),
        in_specs=[
            pl.BlockSpec((bm, H), lambda i: (i, 0)),
            pl.BlockSpec((bm, H), lambda i: (i, 0)),
            pl.BlockSpec((H, H), lambda i: (0, 0)),
            pl.BlockSpec((H,), lambda i: (0,)),
            pl.BlockSpec((H, H), lambda i: (0, 0)),
            pl.BlockSpec((H,), lambda i: (0,)),
            pl.BlockSpec((H,), lambda i: (0,)),
            pl.BlockSpec((H,), lambda i: (0,)),
        ],
        out_specs=pl.BlockSpec((bm, H), lambda i: (i, 0)),
        out_shape=jax.ShapeDtypeStruct((m, h.shape[1]), jnp.float32),
    )(agg, h, w1, b1, w2, b2, g, bb)


def kernel(cloud_x, cloud_batch, node_W1, node_b1, node_W2, node_b2,
           edge_W1, edge_b1, edge_W2, edge_b2, conv_W1, conv_b1, conv_W2,
           conv_b2, ln_g, ln_b):
    n = cloud_x.shape[0]
    xyz = cloud_x[:, :3]
    sq = jnp.sum(xyz * xyz, axis=1)
    batch = cloud_batch.astype(jnp.int32)

    gids = jnp.arange(NGRAPH, dtype=jnp.int32)
    starts = jnp.searchsorted(batch, gids, side="left").astype(jnp.int32)
    ends = jnp.searchsorted(batch, gids, side="right").astype(jnp.int32)

    nblk = NP // BR
    r0 = jnp.arange(nblk, dtype=jnp.int32) * BR
    r1 = jnp.minimum(r0 + BR - 1, n - 1)
    valid = r0 < n
    b0 = batch[jnp.minimum(r0, n - 1)]
    b1 = batch[r1]
    cs = jnp.where(valid, starts[b0] // 128, 0)
    ct = jnp.where(valid, (ends[b1] + 127) // 128 - cs, 1)

    rows = jnp.pad(jnp.concatenate([xyz, sq[:, None]], axis=1),
                   ((0, NP - n), (0, 0)))
    rowsb = jnp.pad(batch[:, None], ((0, NP - n), (0, 0)),
                    constant_values=-2)
    colsT = jnp.concatenate([
        jnp.pad(xyz.T, ((0, 0), (0, NP - n))),
        jnp.pad(sq[None], ((0, 0), (0, NP - n))),
        jnp.pad(batch[None].astype(jnp.float32), ((0, 0), (0, NP - n)),
                constant_values=-1.0),
        jnp.zeros((3, NP), jnp.float32),
    ]).reshape(8, NT, 128).transpose(1, 0, 2)

    idx = _knn_call(cs, ct, rows, rowsb, colsT)[:n]
    idxf = jnp.pad(idx, ((0, NP - n), (0, 0))).reshape(-1)

    xs = xyz[idx]
    delta = xyz[:, None, :] - xs
    dist = jnp.sqrt(jnp.sum(delta * delta, axis=-1, keepdims=True))
    raw_edge = jnp.concatenate(
        [delta, dist, jnp.zeros((n, K, 4), jnp.float32)], axis=-1)

    BM = 128

    ew1 = jnp.concatenate([edge_W1, jnp.zeros((4, H), jnp.float32)], axis=0)
    edge_attr = _mlp2_call(raw_edge.reshape(n * K, 8), ew1, edge_b1,
                           edge_W2, edge_b2, 640)

    xin = jnp.pad(cloud_x, ((0, NP - n), (0, 1)))
    nw1 = jnp.concatenate([node_W1, jnp.zeros((1, H), jnp.float32)], axis=0)
    h = _mlp2_call(xin, nw1, node_b1, node_W2, node_b2, 512)

    for i in range(L):
        agg = _msg_call(h, edge_attr, idxf)
        h = _conv_call(agg, h, conv_W1[i], conv_b1[i], conv_W2[i],
                       conv_b2[i], ln_g[i], ln_b[i], BM)
    return h[:n]
